# baseline (speedup 1.0000x reference)
import functools

import jax
import jax.numpy as jnp
from jax import lax
from jax.experimental import pallas as pl
from jax.experimental.pallas import tpu as pltpu
from jax.experimental.pallas import tpu_sc as plsc

POOL = 7
L = 16

_SPLAT_DNUMS = lax.GatherDimensionNumbers(
    offset_dims=(), collapsed_slice_dims=(0,), start_index_map=(0,)
)


def _splat(vec, rv):
    return lax.gather(
        vec, rv[:, None], dimension_numbers=_SPLAT_DNUMS, slice_sizes=(1,),
        mode=lax.GatherScatterMode.PROMISE_IN_BOUNDS,
    )


def _roi_pool_sc(boxes_t, table, inv_scale, *, n_rois, height, width, channels,
                 chunk_lo=0, chunk_hi=None, out_rois=None):
    NC, NS = 2, 16
    NW = NC * NS
    n_chunks = n_rois // L
    if chunk_hi is None:
        chunk_hi = n_chunks
    if out_rois is None:
        out_rois = n_rois
    my_chunks = chunk_hi - chunk_lo
    base_chunks = my_chunks // NW
    rem = my_chunks % NW
    cpv = channels // L
    JH = (POOL + 1) // 2
    max_rows = JH * L
    row_w = 4 * channels

    mesh = plsc.VectorSubcoreMesh(core_axis_name="c", subcore_axis_name="s")

    @functools.partial(
        pl.kernel,
        mesh=mesh,
        out_type=jax.ShapeDtypeStruct((out_rois, POOL, POOL, channels), jnp.float32),
        scratch_types=[
            pltpu.VMEM((L,), jnp.float32),
            pltpu.VMEM((2, 4 * L), jnp.float32),
            pltpu.VMEM((POOL, L * POOL), jnp.int32),
            pltpu.VMEM((2, max_rows, row_w // 2), jnp.int32),
            pltpu.VMEM((5, POOL, POOL, L), jnp.float32),
            pltpu.VMEM((2, L, POOL, channels), jnp.float32),
            pltpu.SemaphoreType.DMA,
            pltpu.SemaphoreType.DMA,
            pltpu.SemaphoreType.DMA,
            pltpu.SemaphoreType.DMA,
        ],
    )
    def k(boxes_hbm, table_hbm, inv_scale_hbm, out_hbm, inv_s_c, rois_c,
          idx_buf, slab, w_buf, stage, sem0, sem1, sem_out, sem_rois):
        sems = (sem0, sem1)
        wid = lax.axis_index("c") * NS + lax.axis_index("s")
        my_n = base_chunks + jnp.where(wid < rem, 1, 0)
        my_start = chunk_lo + wid * base_chunks + jnp.minimum(wid, rem)
        pltpu.sync_copy(inv_scale_hbm, inv_s_c)
        pltpu.async_copy(
            boxes_hbm.at[pl.ds(my_start * (4 * L), 4 * L)], rois_c.at[0],
            sem_rois,
        )

        def fc(v):
            return jnp.full((L,), v, jnp.float32)

        def ic(v):
            return jnp.full((L,), v, jnp.int32)

        y_hi = float(height - 1)
        x_hi = float(width - 1)

        def chunk_body(kk, carry):
            cid = my_start + kk
            roi0 = cid * L
            pr = lax.bitwise_and(kk, 1)
            pltpu.make_async_copy(
                boxes_hbm.at[pl.ds(cid * (4 * L), 4 * L)], rois_c.at[pr],
                sem_rois,
            ).wait()
            cid_n = jnp.minimum(cid + 1, n_chunks - 1)
            pltpu.async_copy(
                boxes_hbm.at[pl.ds(cid_n * (4 * L), 4 * L)], rois_c.at[1 - pr],
                sem_rois,
            )
            y1 = rois_c[pr, pl.ds(0 * L, L)]
            x1 = rois_c[pr, pl.ds(1 * L, L)]
            y2 = rois_c[pr, pl.ds(2 * L, L)]
            x2 = rois_c[pr, pl.ds(3 * L, L)]
            hs = (y2 - y1) * fc(y_hi / (POOL - 1))
            ws = (x2 - x1) * fc(x_hi / (POOL - 1))
            ybase = y1 * fc(y_hi)
            xbase = x1 * fc(x_hi)

            def axis_coeffs(i_ax, base, step, hi):
                pos = base + fc(float(i_ax)) * step
                valid = jnp.where((pos >= fc(0.0)) & (pos <= fc(hi)), fc(1.0), fc(0.0))
                posc = jnp.minimum(jnp.maximum(pos, fc(0.0)), fc(hi))
                lo = posc.astype(jnp.int32)
                frac = posc - lo.astype(jnp.float32)
                return (fc(1.0) - frac) * valid, frac * valid, lo

            a_w, b_w, tin = [], [], []
            for i_ax in range(POOL):
                wa, wb, lo = axis_coeffs(i_ax, ybase, hs, y_hi)
                a_w.append(wa)
                b_w.append(wb)
                tin.append(lo * ic(width))

            c_w, d_w, li = [], [], []
            for i_ax in range(POOL):
                wc, wd, lo = axis_coeffs(i_ax, xbase, ws, x_hi)
                c_w.append(wc)
                d_w.append(wd)
                li.append(lo)

            inv_s = inv_s_c[...]
            for i in range(POOL):
                a_w[i] = a_w[i] * inv_s
                b_w[i] = b_w[i] * inv_s
            for i in range(POOL):
                ab = a_w[i] + b_w[i]
                for j in range(POOL):
                    idx_buf[i, pl.ds(j * L, L)] = tin[i] + li[j]
                    w_buf[0, i, j, :] = a_w[i] * c_w[j]
                    w_buf[1, i, j, :] = a_w[i] * d_w[j]
                    w_buf[2, i, j, :] = b_w[i] * c_w[j]
                    w_buf[3, i, j, :] = b_w[i] * d_w[j]
                    w_buf[4, i, j, :] = (
                        ab * (c_w[j] + d_w[j]) * fc(32768.0)
                    )

            def unit_rows(jh):
                return max_rows if jh == 0 else POOL * L - max_rows

            def fire(u):
                i, jh = divmod(u, 2)
                par = u % 2
                nr = unit_rows(jh)
                return pltpu.async_copy(
                    table_hbm.at[idx_buf.at[i, pl.ds(jh * max_rows, nr)]],
                    slab.at[par, pl.ds(0, nr)],
                    sems[par],
                )

            handles = {0: fire(0), 1: fire(1)}
            out_h = {}
            for u in range(2 * POOL):
                i, jh = divmod(u, 2)
                par = u % 2
                ip = i % 2
                if jh == 0 and i >= 2:
                    for hdl in out_h.pop(i - 2):
                        hdl.wait()
                handles[u].wait()
                j_lo = 0 if jh == 0 else JH
                nr = unit_rows(jh)

                @plsc.parallel_loop(0, nr, step=1, unroll=4)
                def p_body(p, par=par, ip=ip, i=i, j_lo=j_lo):
                    j_loc = lax.shift_right_logical(p, 4)
                    r = lax.bitwise_and(p, 15)
                    j = j_loc + j_lo
                    rv = lax.broadcast(r, (L,))
                    s_w = [
                        _splat(w_buf[kc, i, j, :], rv) for kc in range(5)
                    ]
                    lomask = jnp.full((L,), 65535, jnp.int32)
                    for h2 in range(cpv // 2):
                        acc0 = None
                        acc1 = None
                        for kc in range(4):
                            bi = kc * 2 + h2
                            va = slab[par, p, pl.ds(bi * L, L)]
                            ua = lax.bitwise_and(va, lomask).astype(jnp.float32)
                            ub = lax.shift_right_logical(va, 16).astype(
                                jnp.float32
                            )
                            if acc0 is None:
                                acc0 = s_w[kc] * ua
                                acc1 = s_w[kc] * ub
                            else:
                                acc0 = acc0 + s_w[kc] * ua
                                acc1 = acc1 + s_w[kc] * ub
                        stage[ip, r, j, pl.ds(h2 * 2 * L, L)] = acc0 - s_w[4]
                        stage[ip, r, j, pl.ds(h2 * 2 * L + L, L)] = acc1 - s_w[4]

                if u + 2 < 2 * POOL:
                    handles[u + 2] = fire(u + 2)
                if jh == 1:
                    out_h[i] = [
                        pltpu.async_copy(
                            stage.at[ip, r],
                            out_hbm.at[roi0 - chunk_lo * L + r, i],
                            sem_out,
                        )
                        for r in range(L)
                    ]
            for i in (POOL - 2, POOL - 1):
                for hdl in out_h.pop(i):
                    hdl.wait()
            return carry

        lax.fori_loop(0, my_n, chunk_body, 0)
        pr_last = lax.bitwise_and(my_n, 1)
        cid_last = jnp.minimum(my_start + my_n, n_chunks - 1)
        pltpu.make_async_copy(
            boxes_hbm.at[pl.ds(cid_last * (4 * L), 4 * L)], rois_c.at[pr_last],
            sem_rois,
        ).wait()

    return k(boxes_t, table, inv_scale)


def kernel(feature, rois, img_size):
    n_rois = rois.shape[0]
    _, height, width, channels = feature.shape
    normalization = jnp.stack(
        [img_size[0], img_size[1], img_size[0], img_size[1]], axis=0
    ).astype(jnp.float32)
    boxes = rois / normalization
    boxes_t = (
        boxes.reshape(n_rois // L, L, 4).transpose(0, 2, 1).reshape(-1)
    )
    img = feature[0]
    xsh = jnp.roll(img, -1, axis=1)
    ysh = jnp.roll(img, -1, axis=0)
    yxsh = jnp.roll(xsh, -1, axis=0)
    table = jnp.concatenate([img, xsh, ysh, yxsh], axis=-1).reshape(
        height * width, 4 * channels
    )
    perm = []
    for blk in range(4 * channels // (2 * L)):
        base = blk * 2 * L
        for m in range(L):
            perm.append(base + m)
            perm.append(base + L + m)
    table = table[:, jnp.array(perm, jnp.int32)]
    scale = 32000.0 / jnp.maximum(jnp.max(jnp.abs(table)), 1e-30)
    q = (jnp.round(table * scale) + 32768.0).astype(jnp.uint16)
    table_q = lax.bitcast_convert_type(
        q.reshape(height * width, 2 * channels, 2), jnp.int32
    )
    inv_scale = jnp.full((L,), 1.0, jnp.float32) / scale
    n_chunks = n_rois // L
    if n_chunks % 2 != 0:
        return _roi_pool_sc(
            boxes_t, table_q, inv_scale,
            n_rois=n_rois, height=height, width=width, channels=channels,
        )
    half = n_chunks // 2
    p0 = _roi_pool_sc(
        boxes_t, table_q, inv_scale,
        n_rois=n_rois, height=height, width=width, channels=channels,
        chunk_lo=0, chunk_hi=half, out_rois=n_rois,
    )
    p1 = _roi_pool_sc(
        boxes_t, table_q, inv_scale,
        n_rois=n_rois, height=height, width=width, channels=channels,
        chunk_lo=half, chunk_hi=n_chunks, out_rois=n_rois - half * L,
    )
    return lax.dynamic_update_slice_in_dim(p0, p1, half * L, axis=0)


# speedup vs baseline: 1.0387x; 1.0387x over previous
import functools

import jax
import jax.numpy as jnp
from jax import lax
from jax.experimental import pallas as pl
from jax.experimental.pallas import tpu as pltpu
from jax.experimental.pallas import tpu_sc as plsc

POOL = 7
L = 16

_SPLAT_DNUMS = lax.GatherDimensionNumbers(
    offset_dims=(), collapsed_slice_dims=(0,), start_index_map=(0,)
)


def _splat(vec, rv):
    return lax.gather(
        vec, rv[:, None], dimension_numbers=_SPLAT_DNUMS, slice_sizes=(1,),
        mode=lax.GatherScatterMode.PROMISE_IN_BOUNDS,
    )


def _roi_pool_sc(boxes_t, table, inv_scale, *, n_rois, height, width, channels):
    NC, NS = 2, 16
    NW = NC * NS
    n_chunks = n_rois // L
    base_chunks = n_chunks // NW
    rem = n_chunks % NW
    cpv = channels // L
    JH = (POOL + 1) // 2
    max_rows = JH * L
    row_w = 4 * channels

    mesh = plsc.VectorSubcoreMesh(core_axis_name="c", subcore_axis_name="s")

    @functools.partial(
        pl.kernel,
        mesh=mesh,
        out_type=jax.ShapeDtypeStruct((n_rois, POOL, POOL, channels), jnp.float32),
        scratch_types=[
            pltpu.VMEM((L,), jnp.float32),
            pltpu.VMEM((2, 4 * L), jnp.float32),
            pltpu.VMEM((POOL, L * POOL), jnp.int32),
            pltpu.VMEM((2, max_rows, row_w // 2), jnp.int32),
            pltpu.VMEM((5, POOL, POOL, L), jnp.float32),
            pltpu.VMEM((2, L, POOL, channels), jnp.float32),
            pltpu.SemaphoreType.DMA,
            pltpu.SemaphoreType.DMA,
            pltpu.SemaphoreType.DMA,
            pltpu.SemaphoreType.DMA,
        ],
    )
    def k(boxes_hbm, table_hbm, inv_scale_hbm, out_hbm, inv_s_c, rois_c,
          idx_buf, slab, w_buf, stage, sem0, sem1, sem_out, sem_rois):
        sems = (sem0, sem1)
        wid = lax.axis_index("c") * NS + lax.axis_index("s")
        my_n = base_chunks + jnp.where(wid < rem, 1, 0)
        my_start = wid * base_chunks + jnp.minimum(wid, rem)
        pltpu.sync_copy(inv_scale_hbm, inv_s_c)
        pltpu.async_copy(
            boxes_hbm.at[pl.ds(my_start * (4 * L), 4 * L)], rois_c.at[0],
            sem_rois,
        )

        def fc(v):
            return jnp.full((L,), v, jnp.float32)

        def ic(v):
            return jnp.full((L,), v, jnp.int32)

        y_hi = float(height - 1)
        x_hi = float(width - 1)

        def chunk_body(kk, carry):
            cid = my_start + kk
            roi0 = cid * L
            pr = lax.bitwise_and(kk, 1)
            pltpu.make_async_copy(
                boxes_hbm.at[pl.ds(cid * (4 * L), 4 * L)], rois_c.at[pr],
                sem_rois,
            ).wait()
            cid_n = jnp.minimum(cid + 1, n_chunks - 1)
            pltpu.async_copy(
                boxes_hbm.at[pl.ds(cid_n * (4 * L), 4 * L)], rois_c.at[1 - pr],
                sem_rois,
            )
            y1 = rois_c[pr, pl.ds(0 * L, L)]
            x1 = rois_c[pr, pl.ds(1 * L, L)]
            y2 = rois_c[pr, pl.ds(2 * L, L)]
            x2 = rois_c[pr, pl.ds(3 * L, L)]
            hs = (y2 - y1) * fc(y_hi / (POOL - 1))
            ws = (x2 - x1) * fc(x_hi / (POOL - 1))
            ybase = y1 * fc(y_hi)
            xbase = x1 * fc(x_hi)

            def axis_coeffs(i_ax, base, step, hi):
                pos = base + fc(float(i_ax)) * step
                valid = jnp.where((pos >= fc(0.0)) & (pos <= fc(hi)), fc(1.0), fc(0.0))
                posc = jnp.minimum(jnp.maximum(pos, fc(0.0)), fc(hi))
                lo = posc.astype(jnp.int32)
                frac = posc - lo.astype(jnp.float32)
                return (fc(1.0) - frac) * valid, frac * valid, lo

            a_w, b_w, tin = [], [], []
            for i_ax in range(POOL):
                wa, wb, lo = axis_coeffs(i_ax, ybase, hs, y_hi)
                a_w.append(wa)
                b_w.append(wb)
                tin.append(lo * ic(width))

            c_w, d_w, li = [], [], []
            for i_ax in range(POOL):
                wc, wd, lo = axis_coeffs(i_ax, xbase, ws, x_hi)
                c_w.append(wc)
                d_w.append(wd)
                li.append(lo)

            inv_s = inv_s_c[...]
            for i in range(POOL):
                a_w[i] = a_w[i] * inv_s
                b_w[i] = b_w[i] * inv_s
            for i in range(POOL):
                ab = a_w[i] + b_w[i]
                for j in range(POOL):
                    idx_buf[i, pl.ds(j * L, L)] = tin[i] + li[j]
                    w_buf[0, i, j, :] = a_w[i] * c_w[j]
                    w_buf[1, i, j, :] = a_w[i] * d_w[j]
                    w_buf[2, i, j, :] = b_w[i] * c_w[j]
                    w_buf[3, i, j, :] = b_w[i] * d_w[j]
                    w_buf[4, i, j, :] = (
                        ab * (c_w[j] + d_w[j]) * fc(32768.0)
                    )

            def unit_rows(jh):
                return max_rows if jh == 0 else POOL * L - max_rows

            def fire(u):
                i, jh = divmod(u, 2)
                par = u % 2
                nr = unit_rows(jh)
                return pltpu.async_copy(
                    table_hbm.at[idx_buf.at[i, pl.ds(jh * max_rows, nr)]],
                    slab.at[par, pl.ds(0, nr)],
                    sems[par],
                )

            handles = {0: fire(0), 1: fire(1)}
            out_h = {}
            for u in range(2 * POOL):
                i, jh = divmod(u, 2)
                par = u % 2
                ip = i % 2
                if jh == 0 and i >= 2:
                    for hdl in out_h.pop(i - 2):
                        hdl.wait()
                handles[u].wait()
                j_lo = 0 if jh == 0 else JH
                nr = unit_rows(jh)

                @plsc.parallel_loop(0, nr, step=1, unroll=4)
                def p_body(p, par=par, ip=ip, i=i, j_lo=j_lo):
                    j_loc = lax.shift_right_logical(p, 4)
                    r = lax.bitwise_and(p, 15)
                    j = j_loc + j_lo
                    rv = lax.broadcast(r, (L,))
                    s_w = [
                        _splat(w_buf[kc, i, j, :], rv) for kc in range(5)
                    ]
                    lomask = jnp.full((L,), 65535, jnp.int32)
                    for h2 in range(cpv // 2):
                        acc0 = None
                        acc1 = None
                        for kc in range(4):
                            bi = kc * 2 + h2
                            va = slab[par, p, pl.ds(bi * L, L)]
                            ua = lax.bitwise_and(va, lomask).astype(jnp.float32)
                            ub = lax.shift_right_logical(va, 16).astype(
                                jnp.float32
                            )
                            if acc0 is None:
                                acc0 = s_w[kc] * ua
                                acc1 = s_w[kc] * ub
                            else:
                                acc0 = acc0 + s_w[kc] * ua
                                acc1 = acc1 + s_w[kc] * ub
                        stage[ip, r, j, pl.ds(h2 * 2 * L, L)] = acc0 - s_w[4]
                        stage[ip, r, j, pl.ds(h2 * 2 * L + L, L)] = acc1 - s_w[4]

                if u + 2 < 2 * POOL:
                    handles[u + 2] = fire(u + 2)
                if jh == 1:
                    out_h[i] = [
                        pltpu.async_copy(
                            stage.at[ip, r], out_hbm.at[roi0 + r, i], sem_out
                        )
                        for r in range(L)
                    ]
            for i in (POOL - 2, POOL - 1):
                for hdl in out_h.pop(i):
                    hdl.wait()
            return carry

        lax.fori_loop(0, my_n, chunk_body, 0)
        pr_last = lax.bitwise_and(my_n, 1)
        cid_last = jnp.minimum(my_start + my_n, n_chunks - 1)
        pltpu.make_async_copy(
            boxes_hbm.at[pl.ds(cid_last * (4 * L), 4 * L)], rois_c.at[pr_last],
            sem_rois,
        ).wait()

    return k(boxes_t, table, inv_scale)


def kernel(feature, rois, img_size):
    n_rois = rois.shape[0]
    _, height, width, channels = feature.shape
    normalization = jnp.stack(
        [img_size[0], img_size[1], img_size[0], img_size[1]], axis=0
    ).astype(jnp.float32)
    boxes = rois / normalization
    boxes_t = (
        boxes.reshape(n_rois // L, L, 4).transpose(0, 2, 1).reshape(-1)
    )
    img = feature[0]
    xsh = jnp.roll(img, -1, axis=1)
    ysh = jnp.roll(img, -1, axis=0)
    yxsh = jnp.roll(xsh, -1, axis=0)
    table = jnp.concatenate([img, xsh, ysh, yxsh], axis=-1).reshape(
        height * width, 4 * channels
    )
    perm = []
    for blk in range(4 * channels // (2 * L)):
        base = blk * 2 * L
        for m in range(L):
            perm.append(base + m)
            perm.append(base + L + m)
    table = table[:, jnp.array(perm, jnp.int32)]
    scale = 32000.0 / jnp.maximum(jnp.max(jnp.abs(table)), 1e-30)
    q = (jnp.round(table * scale) + 32768.0).astype(jnp.uint16)
    table_q = lax.bitcast_convert_type(
        q.reshape(height * width, 2 * channels, 2), jnp.int32
    )
    inv_scale = jnp.full((L,), 1.0, jnp.float32) / scale
    return _roi_pool_sc(
        boxes_t, table_q, inv_scale,
        n_rois=n_rois, height=height, width=width, channels=channels,
    )
